# baseline (speedup 1.0000x reference)
import functools

import jax
import jax.numpy as jnp
from jax import lax
from jax.experimental import pallas as pl
from jax.experimental.pallas import tpu as pltpu
from jax.experimental.pallas import tpu_sc as plsc

N = 10000
E = 320000
D = 128
NC = 2
NS = 16
NW = NC * NS
PW = E // NW
C = 80
CH = PW // C
STRIPE = N // NS

_MESH = plsc.VectorSubcoreMesh(core_axis_name="c", subcore_axis_name="s")


def _deg_body(dst_hbm, zrows_hbm, ones_hbm, out_hbm, dst_blk, ones_v, acc):
    c = lax.axis_index("c")
    s = lax.axis_index("s")
    wid = c * NS + s

    pltpu.sync_copy(zrows_hbm, acc.at[pl.ds(s * STRIPE, STRIPE)])
    pltpu.sync_copy(dst_hbm.at[wid], dst_blk)
    pltpu.sync_copy(ones_hbm, ones_v)
    plsc.subcore_barrier()

    def body(j, _):
        pltpu.sync_copy(ones_v, acc.at[dst_blk.at[j]], add=True)
        return 0

    lax.fori_loop(0, CH, body, 0)
    plsc.subcore_barrier()

    pltpu.sync_copy(acc.at[pl.ds(s * STRIPE, STRIPE)], out_hbm.at[wid])


_deg_call = pl.kernel(
    _deg_body,
    out_type=jax.ShapeDtypeStruct((NW, STRIPE, D), jnp.float32),
    mesh=_MESH,
    scratch_types=[
        pltpu.VMEM((CH, C), jnp.int32),
        pltpu.VMEM((C, D), jnp.float32),
        pltpu.VMEM_SHARED((N, D), jnp.float32),
    ],
)


NBUF = 2


def _edge_body(table_hbm, src_hbm, dst_hbm, zrows_hbm, out_hbm,
               src_blk, dst_blk, r0, r1, g0, g1, t0, t1, acc):
    c = lax.axis_index("c")
    s = lax.axis_index("s")
    wid = c * NS + s
    rows = [r0, r1]
    gsems = [g0, g1]
    ssems = [t0, t1]

    pltpu.sync_copy(zrows_hbm, acc.at[pl.ds(s * STRIPE, STRIPE)])
    pltpu.sync_copy(src_hbm.at[wid], src_blk)
    pltpu.sync_copy(dst_hbm.at[wid], dst_blk)
    plsc.subcore_barrier()

    pltpu.async_copy(table_hbm.at[src_blk.at[pl.ds(0, C)]], rows[0], gsems[0])

    def body(g, _):
        for b in range(NBUF):
            j = g * NBUF + b
            o = 1 - b
            pltpu.make_async_copy(
                table_hbm.at[src_blk.at[pl.ds(j * C, C)]],
                rows[b], gsems[b]).wait()

            @pl.when(j > 0)
            def _():
                pltpu.make_async_copy(
                    rows[o], acc.at[dst_blk.at[j - 1]], ssems[o]).wait()

            @pl.when(j + 1 < CH)
            def _():
                pltpu.async_copy(
                    table_hbm.at[src_blk.at[pl.ds((j + 1) * C, C)]],
                    rows[o], gsems[o])

            pltpu.async_copy(rows[b], acc.at[dst_blk.at[j]], ssems[b],
                             add=True)
        return 0

    lax.fori_loop(0, CH // NBUF, body, 0)
    for j in range((CH // NBUF) * NBUF, CH):
        b = j % NBUF
        o = 1 - b
        pltpu.make_async_copy(
            table_hbm.at[src_blk.at[pl.ds(j * C, C)]], rows[b],
            gsems[b]).wait()
        pltpu.make_async_copy(
            rows[o], acc.at[dst_blk.at[j - 1]], ssems[o]).wait()
        pltpu.async_copy(rows[b], acc.at[dst_blk.at[j]], ssems[b], add=True)
    lastj = CH - 1
    lastb = lastj % NBUF
    pltpu.make_async_copy(
        rows[lastb], acc.at[dst_blk.at[lastj]], ssems[lastb]).wait()
    plsc.subcore_barrier()

    pltpu.sync_copy(acc.at[pl.ds(s * STRIPE, STRIPE)], out_hbm.at[wid])


_edge_call = pl.kernel(
    _edge_body,
    out_type=jax.ShapeDtypeStruct((NW, STRIPE, D), jnp.float32),
    mesh=_MESH,
    scratch_types=[
        pltpu.VMEM((PW,), jnp.int32),
        pltpu.VMEM((CH, C), jnp.int32),
        pltpu.VMEM((C, D), jnp.float32),
        pltpu.VMEM((C, D), jnp.float32),
        pltpu.SemaphoreType.DMA,
        pltpu.SemaphoreType.DMA,
        pltpu.SemaphoreType.DMA,
        pltpu.SemaphoreType.DMA,
        pltpu.VMEM_SHARED((N, D), jnp.float32),
    ],
)


def _mm1_body(x_ref, w_ref, dega_ref, degb_ref, xws_ref, dinv_ref):
    deg = dega_ref[...] + degb_ref[...] + 1.0
    dinv = lax.rsqrt(deg)
    dinv_ref[...] = dinv
    xw = jnp.dot(x_ref[...], w_ref[...], preferred_element_type=jnp.float32)
    xws_ref[...] = xw * dinv


def _bn_mm2_body(a0_ref, a1_ref, xws_ref, dinv_ref, b_ref, g_ref, be_ref,
                 w_ref, out_ref):
    t = (a0_ref[...] + a1_ref[...] + xws_ref[...]) * dinv_ref[...] + b_ref[...]
    m = jnp.mean(t, axis=0, keepdims=True)
    v = jnp.mean((t - m) * (t - m), axis=0, keepdims=True)
    y = (t - m) * lax.rsqrt(v + 1e-5) * g_ref[...] + be_ref[...]
    h = jnp.where(y > 0.0, y, jnp.exp(jnp.minimum(y, 0.0)) - 1.0)
    xw = jnp.dot(h, w_ref[...], preferred_element_type=jnp.float32)
    out_ref[...] = xw * dinv_ref[...]


def _bn_body(a0_ref, a1_ref, xws_ref, dinv_ref, b_ref, g_ref, be_ref, out_ref):
    t = (a0_ref[...] + a1_ref[...] + xws_ref[...]) * dinv_ref[...] + b_ref[...]
    m = jnp.mean(t, axis=0, keepdims=True)
    v = jnp.mean((t - m) * (t - m), axis=0, keepdims=True)
    y = (t - m) * lax.rsqrt(v + 1e-5) * g_ref[...] + be_ref[...]
    out_ref[...] = jnp.where(y > 0.0, y, jnp.exp(jnp.minimum(y, 0.0)) - 1.0)


@jax.jit
def kernel(x, edge_index, W1, b1, g1, be1, W2, b2, g2, be2):
    src = edge_index[0].astype(jnp.int32)
    dst = edge_index[1].astype(jnp.int32)
    src2d = src.reshape(NW, PW)
    dst2d = dst.reshape(NW, CH, C)

    zrows = jnp.zeros((STRIPE, D), jnp.float32)
    b1r, g1r, be1r = b1.reshape(1, D), g1.reshape(1, D), be1.reshape(1, D)
    b2r, g2r, be2r = b2.reshape(1, D), g2.reshape(1, D), be2.reshape(1, D)

    onesr = jnp.ones((C, D), jnp.float32)
    degp = _deg_call(dst2d, zrows, onesr)
    degp = degp.reshape(NC, N, D)[:, :, :1]

    xws1, dinv = pl.pallas_call(
        _mm1_body,
        out_shape=[
            jax.ShapeDtypeStruct((N, D), jnp.float32),
            jax.ShapeDtypeStruct((N, 1), jnp.float32),
        ],
    )(x, W1, degp[0], degp[1])

    parts1 = _edge_call(xws1, src2d, dst2d, zrows).reshape(NC, N, D)
    xws2 = pl.pallas_call(
        _bn_mm2_body,
        out_shape=jax.ShapeDtypeStruct((N, D), jnp.float32),
    )(parts1[0], parts1[1], xws1, dinv, b1r, g1r, be1r, W2)

    parts2 = _edge_call(xws2, src2d, dst2d, zrows).reshape(NC, N, D)
    h2 = pl.pallas_call(
        _bn_body,
        out_shape=jax.ShapeDtypeStruct((N, D), jnp.float32),
    )(parts2[0], parts2[1], xws2, dinv, b2r, g2r, be2r)
    return h2


# speedup vs baseline: 1.1717x; 1.1717x over previous
import functools

import jax
import jax.numpy as jnp
from jax import lax
from jax.experimental import pallas as pl
from jax.experimental.pallas import tpu as pltpu
from jax.experimental.pallas import tpu_sc as plsc

N = 10000
E = 320000
D = 128
NC = 2
NS = 16
NW = NC * NS
PW = E // NW
C = 80
CH = PW // C
STRIPE = N // NS

_MESH = plsc.VectorSubcoreMesh(core_axis_name="c", subcore_axis_name="s")


def _deg_body(dst_hbm, zrows_hbm, ones_hbm, out_hbm, dst_blk, ones_v, acc):
    c = lax.axis_index("c")
    s = lax.axis_index("s")
    wid = c * NS + s

    pltpu.sync_copy(zrows_hbm, acc.at[pl.ds(s * STRIPE, STRIPE)])
    pltpu.sync_copy(dst_hbm.at[wid], dst_blk)
    pltpu.sync_copy(ones_hbm, ones_v)
    plsc.subcore_barrier()

    def body(j, _):
        pltpu.sync_copy(ones_v, acc.at[dst_blk.at[j]], add=True)
        return 0

    lax.fori_loop(0, CH, body, 0)
    plsc.subcore_barrier()

    pltpu.sync_copy(acc.at[pl.ds(s * STRIPE, STRIPE)], out_hbm.at[wid])


_deg_call = pl.kernel(
    _deg_body,
    out_type=jax.ShapeDtypeStruct((NW, STRIPE, D), jnp.float32),
    mesh=_MESH,
    scratch_types=[
        pltpu.VMEM((CH, C), jnp.int32),
        pltpu.VMEM((C, D), jnp.float32),
        pltpu.VMEM_SHARED((N, D), jnp.float32),
    ],
)


NBUF = 2


def _edge_body(table_hbm, src_hbm, dst_hbm, zrows_hbm, out_hbm,
               src_blk, dst_blk, r0, r1, s0, s1, acc):
    c = lax.axis_index("c")
    s = lax.axis_index("s")
    wid = c * NS + s
    rows = [r0, r1]
    sems = [s0, s1]

    pltpu.sync_copy(zrows_hbm, acc.at[pl.ds(s * STRIPE, STRIPE)])
    pltpu.sync_copy(src_hbm.at[wid], src_blk)
    pltpu.sync_copy(dst_hbm.at[wid], dst_blk)
    plsc.subcore_barrier()

    for b in range(NBUF):
        pltpu.async_copy(
            table_hbm.at[src_blk.at[pl.ds(b * C, C)]], rows[b], sems[b])

    def body(g, _):
        j0 = g * NBUF
        for b in range(NBUF):
            j = j0 + b
            pltpu.make_async_copy(
                table_hbm.at[src_blk.at[pl.ds(j * C, C)]],
                rows[b], sems[b]).wait()
            pltpu.sync_copy(rows[b], acc.at[dst_blk.at[j]], add=True)

            @pl.when(j + NBUF < CH)
            def _():
                pltpu.async_copy(
                    table_hbm.at[src_blk.at[pl.ds((j + NBUF) * C, C)]],
                    rows[b], sems[b])
        return 0

    lax.fori_loop(0, CH // NBUF, body, 0)
    for j in range((CH // NBUF) * NBUF, CH):
        b = j % NBUF
        pltpu.make_async_copy(
            table_hbm.at[src_blk.at[pl.ds(j * C, C)]], rows[b], sems[b]).wait()
        pltpu.sync_copy(rows[b], acc.at[dst_blk.at[j]], add=True)
    plsc.subcore_barrier()

    pltpu.sync_copy(acc.at[pl.ds(s * STRIPE, STRIPE)], out_hbm.at[wid])


_edge_call = pl.kernel(
    _edge_body,
    out_type=jax.ShapeDtypeStruct((NW, STRIPE, D), jnp.float32),
    mesh=_MESH,
    scratch_types=[
        pltpu.VMEM((PW,), jnp.int32),
        pltpu.VMEM((CH, C), jnp.int32),
        pltpu.VMEM((C, D), jnp.float32),
        pltpu.VMEM((C, D), jnp.float32),
        pltpu.SemaphoreType.DMA,
        pltpu.SemaphoreType.DMA,
        pltpu.VMEM_SHARED((N, D), jnp.float32),
    ],
)


def _mm1_body(x_ref, w_ref, dega_ref, degb_ref, xws_ref, dinv_ref):
    deg = dega_ref[...] + degb_ref[...] + 1.0
    dinv = lax.rsqrt(deg)
    dinv_ref[...] = dinv
    xw = jnp.dot(x_ref[...], w_ref[...], preferred_element_type=jnp.float32)
    xws_ref[...] = xw * dinv


def _bn_mm2_body(a0_ref, a1_ref, xws_ref, dinv_ref, b_ref, g_ref, be_ref,
                 w_ref, out_ref):
    t = (a0_ref[...] + a1_ref[...] + xws_ref[...]) * dinv_ref[...] + b_ref[...]
    m = jnp.mean(t, axis=0, keepdims=True)
    v = jnp.mean((t - m) * (t - m), axis=0, keepdims=True)
    y = (t - m) * lax.rsqrt(v + 1e-5) * g_ref[...] + be_ref[...]
    h = jnp.where(y > 0.0, y, jnp.exp(jnp.minimum(y, 0.0)) - 1.0)
    xw = jnp.dot(h, w_ref[...], preferred_element_type=jnp.float32)
    out_ref[...] = xw * dinv_ref[...]


def _bn_body(a0_ref, a1_ref, xws_ref, dinv_ref, b_ref, g_ref, be_ref, out_ref):
    t = (a0_ref[...] + a1_ref[...] + xws_ref[...]) * dinv_ref[...] + b_ref[...]
    m = jnp.mean(t, axis=0, keepdims=True)
    v = jnp.mean((t - m) * (t - m), axis=0, keepdims=True)
    y = (t - m) * lax.rsqrt(v + 1e-5) * g_ref[...] + be_ref[...]
    out_ref[...] = jnp.where(y > 0.0, y, jnp.exp(jnp.minimum(y, 0.0)) - 1.0)


@jax.jit
def kernel(x, edge_index, W1, b1, g1, be1, W2, b2, g2, be2):
    src = edge_index[0].astype(jnp.int32)
    dst = edge_index[1].astype(jnp.int32)
    src2d = src.reshape(NW, PW)
    dst2d = dst.reshape(NW, CH, C)

    zrows = jnp.zeros((STRIPE, D), jnp.float32)
    b1r, g1r, be1r = b1.reshape(1, D), g1.reshape(1, D), be1.reshape(1, D)
    b2r, g2r, be2r = b2.reshape(1, D), g2.reshape(1, D), be2.reshape(1, D)

    onesr = jnp.ones((C, D), jnp.float32)
    degp = _deg_call(dst2d, zrows, onesr)
    degp = degp.reshape(NC, N, D)[:, :, :1]

    xws1, dinv = pl.pallas_call(
        _mm1_body,
        out_shape=[
            jax.ShapeDtypeStruct((N, D), jnp.float32),
            jax.ShapeDtypeStruct((N, 1), jnp.float32),
        ],
    )(x, W1, degp[0], degp[1])

    parts1 = _edge_call(xws1, src2d, dst2d, zrows).reshape(NC, N, D)
    xws2 = pl.pallas_call(
        _bn_mm2_body,
        out_shape=jax.ShapeDtypeStruct((N, D), jnp.float32),
    )(parts1[0], parts1[1], xws1, dinv, b1r, g1r, be1r, W2)

    parts2 = _edge_call(xws2, src2d, dst2d, zrows).reshape(NC, N, D)
    h2 = pl.pallas_call(
        _bn_body,
        out_shape=jax.ShapeDtypeStruct((N, D), jnp.float32),
    )(parts2[0], parts2[1], xws2, dinv, b2r, g2r, be2r)
    return h2
